# baseline (speedup 1.0000x reference)
import jax
import jax.numpy as jnp
from jax import lax
from jax.experimental import pallas as pl
from jax.experimental.pallas import tpu as pltpu
from jax.experimental.pallas import tpu_sc as plsc
import functools

TOKENS = 100
DIM = 1024
BATCH = 1024
B = BATCH * TOKENS

NC, NS = 2, 16
NW = NC * NS
B_PER_W = B // NW
L = 16
NSTEPS = B_PER_W // L


def _make_kernel():
    mesh = plsc.VectorSubcoreMesh(core_axis_name="c", subcore_axis_name="s")

    @functools.partial(
        pl.kernel,
        out_type=jax.ShapeDtypeStruct((TOKENS, BATCH, DIM), jnp.float32),
        mesh=mesh,
        scratch_types=[
            pltpu.VMEM((TOKENS, DIM), jnp.float32),
            pltpu.VMEM((B_PER_W,), jnp.int32),
            pltpu.SemaphoreType.DMA,
        ],
        compiler_params=pltpu.CompilerParams(use_tc_tiling_on_sc=True),
    )
    def emb(idx_hbm, table_hbm, out_hbm, table_v, idx_v, ssem):
        wid = lax.axis_index("s") * NC + lax.axis_index("c")
        base = wid * B_PER_W
        pltpu.sync_copy(table_hbm, table_v)
        pltpu.sync_copy(idx_hbm.at[pl.ds(base, B_PER_W)], idx_v)

        def fire_step(ci, carry):
            goff = base + ci * L
            vec = idx_v[pl.ds(ci * L, L)]
            for j in range(L):
                i = jnp.squeeze(lax.slice(vec, (j,), (j + 1,)))
                g = goff + j
                pltpu.make_async_copy(
                    table_v.at[i], out_hbm.at[g // BATCH, g % BATCH], ssem
                ).start()
            return carry

        lax.fori_loop(0, NSTEPS, fire_step, 0)

        def drain_step(r, carry):
            pltpu.make_async_copy(
                table_v.at[0], out_hbm.at[0, 0], ssem
            ).wait()
            return carry

        lax.fori_loop(0, B_PER_W, drain_step, 0)

    return emb


_emb = _make_kernel()


@jax.jit
def kernel(indices, embedding_weight):
    idx_t = indices.T.reshape(B).astype(jnp.int32)
    out_tbd = _emb(idx_t, embedding_weight)
    return out_tbd.transpose(1, 0, 2)


# speedup vs baseline: 1.0971x; 1.0971x over previous
import jax
import jax.numpy as jnp
from jax import lax
from jax.experimental import pallas as pl
from jax.experimental.pallas import tpu as pltpu
from jax.experimental.pallas import tpu_sc as plsc
import functools

TOKENS = 100
DIM = 1024
BATCH = 1024
B = BATCH * TOKENS

NC, NS = 2, 16
NW = NC * NS
B_PER_W = B // NW
L = 16
NSTEPS = B_PER_W // L


def _make_kernel():
    mesh = plsc.VectorSubcoreMesh(core_axis_name="c", subcore_axis_name="s")

    @functools.partial(
        pl.kernel,
        out_type=jax.ShapeDtypeStruct((TOKENS, BATCH, DIM), jnp.float32),
        mesh=mesh,
        scratch_types=[
            pltpu.VMEM((TOKENS, DIM), jnp.float32),
            pltpu.VMEM((B_PER_W,), jnp.int32),
            pltpu.SemaphoreType.DMA,
        ],
        compiler_params=pltpu.CompilerParams(use_tc_tiling_on_sc=True),
    )
    def emb(idx_hbm, table_hbm, out_hbm, table_v, idx_v, ssem):
        wid = lax.axis_index("s") * NC + lax.axis_index("c")
        base = wid * B_PER_W
        pltpu.sync_copy(table_hbm, table_v)
        pltpu.sync_copy(idx_hbm.at[pl.ds(base, B_PER_W)], idx_v)

        def fire_step(ci, carry):
            goff = base + ci * L
            vec = idx_v[pl.ds(ci * L, L)]
            for j in range(L):
                i = jnp.squeeze(lax.slice(vec, (j,), (j + 1,)))
                g = goff + j
                pltpu.make_async_copy(
                    table_v.at[i], out_hbm.at[g // BATCH, g % BATCH], ssem
                ).start()
            return carry

        lax.fori_loop(0, NSTEPS, fire_step, 0)

        def drain_step(r, carry):
            pltpu.make_async_copy(
                table_v.at[pl.ds(0, 64)],
                out_hbm.at[0, pl.ds(0, 64)],
                ssem,
            ).wait()
            return carry

        lax.fori_loop(0, B_PER_W // 64, drain_step, 0)

    return emb


_emb = _make_kernel()


@jax.jit
def kernel(indices, embedding_weight):
    idx_t = indices.T.reshape(B).astype(jnp.int32)
    out_tbd = _emb(idx_t, embedding_weight)
    return out_tbd.transpose(1, 0, 2)
